# baseline (speedup 1.0000x reference)
import functools
import jax
import jax.numpy as jnp
from jax import lax
from jax.experimental import pallas as pl
from jax.experimental.pallas import tpu as pltpu, tpu_sc as plsc

MAXLEN_ = 8192
DIM_ = 768
BATCH_ = 4
NTOT = BATCH_ * MAXLEN_ * DIM_
POS_N = MAXLEN_ * DIM_
NW = 32
EPW = NTOT // NW
CE = 24576
NCHUNK = EPW // CE


def _sc_add(x_hbm, pos_hbm, out_hbm, xbuf, pbuf, sem):
    wid = lax.axis_index("s") * 2 + lax.axis_index("c")
    base_w = wid * EPW
    pos_w = lax.rem(base_w, POS_N)

    def chunk_body(c, _):
        xbase = base_w + c * CE
        pbase = pos_w + c * CE
        pltpu.sync_copy(x_hbm.at[pl.ds(xbase, CE)], xbuf)
        pltpu.sync_copy(pos_hbm.at[pl.ds(pbase, CE)], pbuf)

        @plsc.parallel_loop(0, CE, 16, unroll=8)
        def add_body(i):
            plsc.addupdate(xbuf.at[pl.ds(i, 16)], pbuf[pl.ds(i, 16)])

        pltpu.sync_copy(xbuf, out_hbm.at[pl.ds(xbase, CE)])
        return 0

    lax.fori_loop(0, NCHUNK, chunk_body, 0)


def kernel(x, pos_table):
    xf = x.reshape(-1)
    pf = pos_table.reshape(-1)
    mesh = plsc.VectorSubcoreMesh(
        core_axis_name="c", subcore_axis_name="s", num_cores=2, num_subcores=16
    )
    run = pl.kernel(
        _sc_add,
        out_type=jax.ShapeDtypeStruct((NTOT,), jnp.float32),
        mesh=mesh,
        scratch_types=[
            pltpu.VMEM((CE,), jnp.float32),
            pltpu.VMEM((CE,), jnp.float32),
            pltpu.SemaphoreType.DMA,
        ],
    )
    out = run(xf, pf)
    return out.reshape(x.shape)


# speedup vs baseline: 1.2406x; 1.2406x over previous
import jax
import jax.numpy as jnp
from jax import lax
from jax.experimental import pallas as pl
from jax.experimental.pallas import tpu as pltpu, tpu_sc as plsc

MAXLEN_ = 8192
DIM_ = 768
BATCH_ = 4
NTOT = BATCH_ * MAXLEN_ * DIM_
POS_N = MAXLEN_ * DIM_
NW = 32
EPW = NTOT // NW
CE = 24576
NCHUNK = EPW // CE


def _sc_add(x_hbm, pos_hbm, out_hbm,
            xb0, xb1, pb0, pb1, sx0, sx1, sp0, sp1, so0, so1):
    wid = lax.axis_index("s") * 2 + lax.axis_index("c")
    base_w = wid * EPW
    pos_w = lax.rem(base_w, POS_N)
    xb = (xb0, xb1)
    pb = (pb0, pb1)
    sx = (sx0, sx1)
    sp = (sp0, sp1)
    so = (so0, so1)

    def start_in(c):
        b = c & 1
        xd = pltpu.async_copy(x_hbm.at[pl.ds(base_w + c * CE, CE)], xb[b], sx[b])
        pd = pltpu.async_copy(pos_hbm.at[pl.ds(pos_w + c * CE, CE)], pb[b], sp[b])
        return xd, pd

    ind = {0: start_in(0)}
    outd = {}
    for c in range(NCHUNK):
        b = c & 1
        if 1 <= c < NCHUNK - 1:
            outd[c - 1].wait()
        if c + 1 < NCHUNK:
            ind[c + 1] = start_in(c + 1)
        xd, pd = ind[c]
        xd.wait()
        pd.wait()

        @plsc.parallel_loop(0, CE, 16, unroll=8)
        def add_body(i):
            plsc.addupdate(xb[b].at[pl.ds(i, 16)], pb[b][pl.ds(i, 16)])

        outd[c] = pltpu.async_copy(
            xb[b], out_hbm.at[pl.ds(base_w + c * CE, CE)], so[b])

    outd[NCHUNK - 2].wait()
    outd[NCHUNK - 1].wait()


def kernel(x, pos_table):
    xf = x.reshape(-1)
    pf = pos_table.reshape(-1)
    mesh = plsc.VectorSubcoreMesh(
        core_axis_name="c", subcore_axis_name="s", num_cores=2, num_subcores=16
    )
    run = pl.kernel(
        _sc_add,
        out_type=jax.ShapeDtypeStruct((NTOT,), jnp.float32),
        mesh=mesh,
        scratch_types=[
            pltpu.VMEM((CE,), jnp.float32),
            pltpu.VMEM((CE,), jnp.float32),
            pltpu.VMEM((CE,), jnp.float32),
            pltpu.VMEM((CE,), jnp.float32),
            pltpu.SemaphoreType.DMA,
            pltpu.SemaphoreType.DMA,
            pltpu.SemaphoreType.DMA,
            pltpu.SemaphoreType.DMA,
            pltpu.SemaphoreType.DMA,
            pltpu.SemaphoreType.DMA,
        ],
    )
    out = run(xf, pf)
    return out.reshape(x.shape)
